# baseline (speedup 1.0000x reference)
import jax
import jax.numpy as jnp
from jax.experimental import pallas as pl


def _softmax_row_kernel(x_ref, o_ref):
    x = x_ref[...]
    m = jnp.max(x)
    e = jnp.exp(x - m)
    s = jnp.sum(e)
    o_ref[...] = e * (1.0 / s)


def kernel(logits):
    n_rows, n_cols = logits.shape
    sub = 8
    x = logits.reshape(n_rows * sub, n_cols // sub)
    out = pl.pallas_call(
        _softmax_row_kernel,
        grid=(n_rows,),
        in_specs=[pl.BlockSpec((sub, n_cols // sub), lambda i: (i, 0))],
        out_specs=pl.BlockSpec((sub, n_cols // sub), lambda i: (i, 0)),
        out_shape=jax.ShapeDtypeStruct(x.shape, x.dtype),
    )(x)
    return out.reshape(n_rows, n_cols)
